# baseline (speedup 1.0000x reference)
import functools

import jax
import jax.numpy as jnp
from jax import lax
from jax.experimental import pallas as pl
from jax.experimental.pallas import tpu as pltpu
from jax.experimental.pallas import tpu_sc as plsc

_B = 16384
_D = 16
_ROWS_PER_LINE = 8
_CHUNK = 128
_BLK = 4096


def _sc_gather_lines(u_idx, i_idx, u_lines, i_lines):
    info = plsc.get_sparse_core_info()
    nw = info.num_cores * info.num_subcores
    b_per_w = _B // nw
    n_chunks = b_per_w // _CHUNK
    n_grp = _CHUNK // info.num_lanes

    u_idx_r = u_idx.reshape(nw, n_chunks, _CHUNK)
    i_idx_r = i_idx.reshape(nw, n_chunks, _CHUNK)

    mesh = plsc.VectorSubcoreMesh(core_axis_name="c", subcore_axis_name="s")
    out_sds = jax.ShapeDtypeStruct((nw, n_chunks, _CHUNK, 128), jnp.float32)

    @functools.partial(
        pl.kernel,
        mesh=mesh,
        out_type=(out_sds, out_sds),
        scratch_types=[
            pltpu.VMEM((n_chunks, _CHUNK), jnp.int32),
            pltpu.VMEM((n_chunks, _CHUNK), jnp.int32),
            pltpu.VMEM((n_chunks, _CHUNK), jnp.int32),
            pltpu.VMEM((n_chunks, _CHUNK), jnp.int32),
            pltpu.VMEM((2, _CHUNK, 128), jnp.float32),
            pltpu.VMEM((2, _CHUNK, 128), jnp.float32),
            pltpu.SemaphoreType.DMA,
            pltpu.SemaphoreType.DMA,
            pltpu.SemaphoreType.DMA,
            pltpu.SemaphoreType.DMA,
        ],
    )
    def gather_kernel(u_tab, i_tab, u_idx_hbm, i_idx_hbm, u_out, i_out,
                      uidx_v, iidx_v, uline_v, iline_v, ubuf, ibuf,
                      sem_u0, sem_u1, sem_i0, sem_i1):
        wid = lax.axis_index("s") * info.num_cores + lax.axis_index("c")
        pltpu.sync_copy(u_idx_hbm.at[wid], uidx_v)
        pltpu.sync_copy(i_idx_hbm.at[wid], iidx_v)
        L = info.num_lanes
        for j in range(n_chunks):
            for g in range(n_grp):
                sl = pl.ds(g * L, L)
                uline_v[j, sl] = lax.shift_right_logical(uidx_v[j, sl], 3)
                iline_v[j, sl] = lax.shift_right_logical(iidx_v[j, sl], 3)
        sems_u = (sem_u0, sem_u1)
        sems_i = (sem_i0, sem_i1)

        def fire(j):
            p = j % 2
            cu = pltpu.async_copy(u_tab.at[uline_v.at[j]], ubuf.at[p], sems_u[p])
            ci = pltpu.async_copy(i_tab.at[iline_v.at[j]], ibuf.at[p], sems_i[p])
            return cu, ci

        pending = fire(0)
        for j in range(n_chunks):
            cu, ci = pending
            cu.wait()
            ci.wait()
            if j + 1 < n_chunks:
                pending = fire(j + 1)
            p = j % 2
            pltpu.sync_copy(ubuf.at[p], u_out.at[wid, j])
            pltpu.sync_copy(ibuf.at[p], i_out.at[wid, j])

    u_g, i_g = gather_kernel(u_lines, i_lines, u_idx_r, i_idx_r)
    return u_g.reshape(_B, 128), i_g.reshape(_B, 128)


def _mlp_body(u_ref, i_ref, uix_ref, iix_ref, w1u_ref, w1i_ref, b1_ref,
              w2_ref, b2_ref, w3_ref, b3_ref, o_ref):
    sub_u = lax.bitwise_and(uix_ref[...], 7)
    sub_i = lax.bitwise_and(iix_ref[...], 7)
    zeros = jnp.zeros((_BLK, _D), jnp.float32)
    hu = zeros
    hi = zeros
    for s in range(_ROWS_PER_LINE):
        sl = pl.ds(s * _D, _D)
        hu = hu + jnp.where(sub_u == s, u_ref[:, sl], zeros)
        hi = hi + jnp.where(sub_i == s, i_ref[:, sl], zeros)
    h = (jnp.dot(hu, w1u_ref[...], preferred_element_type=jnp.float32)
         + jnp.dot(hi, w1i_ref[...], preferred_element_type=jnp.float32)
         + b1_ref[...])
    h = jnp.maximum(h, 0.0)
    h = jnp.dot(h, w2_ref[...], preferred_element_type=jnp.float32) + b2_ref[...]
    h = jnp.maximum(h, 0.0)
    z = jnp.sum(h * w3_ref[...], axis=1, keepdims=True) + b3_ref[...]
    o_ref[...] = 1.0 / (1.0 + jnp.exp(-z))


def _tc_mlp(u_g, i_g, u_idx, i_idx, W1, b1, W2, b2, W3, b3):
    w1u = W1[:_D, :]
    w1i = W1[_D:, :]
    b1r = b1.reshape(1, -1)
    b2r = b2.reshape(1, -1)
    w3r = W3.reshape(1, -1)
    b3r = b3.reshape(1, 1)
    uix = u_idx.reshape(_B, 1)
    iix = i_idx.reshape(_B, 1)
    nb = _B // _BLK
    blk = lambda r, c: pl.BlockSpec((r, c), lambda b: (b, 0))
    full = lambda a: pl.BlockSpec(a.shape, lambda b: (0,) * a.ndim)
    out = pl.pallas_call(
        _mlp_body,
        grid=(nb,),
        in_specs=[
            blk(_BLK, 128), blk(_BLK, 128), blk(_BLK, 1), blk(_BLK, 1),
            full(w1u), full(w1i), full(b1r), full(W2), full(b2r),
            full(w3r), full(b3r),
        ],
        out_specs=blk(_BLK, 1),
        out_shape=jax.ShapeDtypeStruct((_B, 1), jnp.float32),
    )(u_g, i_g, uix, iix, w1u, w1i, b1r, W2, b2r, w3r, b3r)
    return out.reshape(-1)


def kernel(u_idx, i_idx, u_table, i_table, W1, b1, W2, b2, W3, b3):
    u_lines = u_table.reshape(-1, 128)
    i_lines = i_table.reshape(-1, 128)
    u_g, i_g = _sc_gather_lines(u_idx, i_idx, u_lines, i_lines)
    return _tc_mlp(u_g, i_g, u_idx, i_idx, W1, b1, W2, b2, W3, b3)


# speedup vs baseline: 1.3058x; 1.3058x over previous
import functools

import jax
import jax.numpy as jnp
from jax import lax
from jax.experimental import pallas as pl
from jax.experimental.pallas import tpu as pltpu
from jax.experimental.pallas import tpu_sc as plsc

_B = 16384
_D = 16
_N = 1000000
_PLANES = 8
_NLINES = 131072
_LB = 1024
_CHUNK = 128
_BLK = 2048


def _transform_body(*refs):
    xu = refs[:_PLANES]
    xi = refs[_PLANES:2 * _PLANES]
    wu_ref, wi_ref, yu_ref, yi_ref = refs[2 * _PLANES:]
    dn = (((0,), (0,)), ((), ()))
    for s in range(_PLANES):
        sl = pl.ds(s * _D, _D)
        yu_ref[:, sl] = lax.dot_general(xu[s][...], wu_ref[...], dn,
                                        preferred_element_type=jnp.float32)
        yi_ref[:, sl] = lax.dot_general(xi[s][...], wi_ref[...], dn,
                                        preferred_element_type=jnp.float32)


def _tc_transform(u_table, i_table, w1u, w1i):
    xu = u_table.T
    xi = i_table.T
    n_blocks = _NLINES // _LB
    bpp = _NLINES // _LB
    last_blk = (_N - 1) // _LB

    def spec(s):
        return pl.BlockSpec(
            (_D, _LB), lambda b, s=s: (0, jnp.minimum(s * bpp + b, last_blk)))

    yu, yi = pl.pallas_call(
        _transform_body,
        grid=(n_blocks,),
        in_specs=([spec(s) for s in range(_PLANES)]
                  + [spec(s) for s in range(_PLANES)]
                  + [pl.BlockSpec((_D, _D), lambda b: (0, 0))] * 2),
        out_specs=[pl.BlockSpec((_LB, 128), lambda b: (b, 0))] * 2,
        out_shape=[jax.ShapeDtypeStruct((_NLINES, 128), jnp.float32)] * 2,
    )(*([xu] * _PLANES + [xi] * _PLANES + [w1u, w1i]))
    return yu, yi


def _sc_gather_lines(u_idx, i_idx, u_lines, i_lines):
    info = plsc.get_sparse_core_info()
    nw = info.num_cores * info.num_subcores
    b_per_w = _B // nw
    n_chunks = b_per_w // _CHUNK
    n_grp = _CHUNK // info.num_lanes

    u_idx_r = u_idx.reshape(nw, n_chunks, _CHUNK)
    i_idx_r = i_idx.reshape(nw, n_chunks, _CHUNK)

    mesh = plsc.VectorSubcoreMesh(core_axis_name="c", subcore_axis_name="s")
    out_sds = jax.ShapeDtypeStruct((nw, n_chunks, _CHUNK, 128), jnp.float32)

    @functools.partial(
        pl.kernel,
        mesh=mesh,
        out_type=(out_sds, out_sds),
        scratch_types=[
            pltpu.VMEM((n_chunks, _CHUNK), jnp.int32),
            pltpu.VMEM((n_chunks, _CHUNK), jnp.int32),
            pltpu.VMEM((n_chunks, _CHUNK), jnp.int32),
            pltpu.VMEM((n_chunks, _CHUNK), jnp.int32),
            pltpu.VMEM((2, _CHUNK, 128), jnp.float32),
            pltpu.VMEM((2, _CHUNK, 128), jnp.float32),
            pltpu.SemaphoreType.DMA,
            pltpu.SemaphoreType.DMA,
            pltpu.SemaphoreType.DMA,
            pltpu.SemaphoreType.DMA,
        ],
    )
    def gather_kernel(u_tab, i_tab, u_idx_hbm, i_idx_hbm, u_out, i_out,
                      uidx_v, iidx_v, uline_v, iline_v, ubuf, ibuf,
                      sem_u0, sem_u1, sem_i0, sem_i1):
        wid = lax.axis_index("s") * info.num_cores + lax.axis_index("c")
        pltpu.sync_copy(u_idx_hbm.at[wid], uidx_v)
        pltpu.sync_copy(i_idx_hbm.at[wid], iidx_v)
        L = info.num_lanes
        for j in range(n_chunks):
            for g in range(n_grp):
                sl = pl.ds(g * L, L)
                uline_v[j, sl] = lax.bitwise_and(uidx_v[j, sl], _NLINES - 1)
                iline_v[j, sl] = lax.bitwise_and(iidx_v[j, sl], _NLINES - 1)
        sems_u = (sem_u0, sem_u1)
        sems_i = (sem_i0, sem_i1)

        def fire(j):
            p = j % 2
            cu = pltpu.async_copy(u_tab.at[uline_v.at[j]], ubuf.at[p], sems_u[p])
            ci = pltpu.async_copy(i_tab.at[iline_v.at[j]], ibuf.at[p], sems_i[p])
            return cu, ci

        pending = fire(0)
        for j in range(n_chunks):
            cu, ci = pending
            cu.wait()
            ci.wait()
            if j + 1 < n_chunks:
                pending = fire(j + 1)
            p = j % 2
            pltpu.sync_copy(ubuf.at[p], u_out.at[wid, j])
            pltpu.sync_copy(ibuf.at[p], i_out.at[wid, j])

    u_g, i_g = gather_kernel(u_lines, i_lines, u_idx_r, i_idx_r)
    return u_g.reshape(_B, 128), i_g.reshape(_B, 128)


def _mlp_body(u_ref, i_ref, uix_ref, iix_ref, b1_ref, w2_ref, b2_ref,
              w3_ref, b3_ref, o_ref):
    sub_u = lax.shift_right_logical(uix_ref[...], 17)
    sub_i = lax.shift_right_logical(iix_ref[...], 17)
    zeros = jnp.zeros((_BLK, _D), jnp.float32)
    h = b1_ref[...] + zeros
    for s in range(_PLANES):
        sl = pl.ds(s * _D, _D)
        h = h + jnp.where(sub_u == s, u_ref[:, sl], zeros)
        h = h + jnp.where(sub_i == s, i_ref[:, sl], zeros)
    h = jnp.maximum(h, 0.0)
    h = jnp.dot(h, w2_ref[...], preferred_element_type=jnp.float32) + b2_ref[...]
    h = jnp.maximum(h, 0.0)
    z = jnp.sum(h * w3_ref[...], axis=1, keepdims=True) + b3_ref[...]
    o_ref[...] = 1.0 / (1.0 + jnp.exp(-z))


def _tc_mlp(u_g, i_g, u_idx, i_idx, b1, W2, b2, W3, b3):
    b1r = b1.reshape(1, -1)
    b2r = b2.reshape(1, -1)
    w3r = W3.reshape(1, -1)
    b3r = b3.reshape(1, 1)
    uix = u_idx.reshape(_B, 1)
    iix = i_idx.reshape(_B, 1)
    nb = _B // _BLK
    blk = lambda r, c: pl.BlockSpec((r, c), lambda b: (b, 0))
    full = lambda a: pl.BlockSpec(a.shape, lambda b: (0,) * a.ndim)
    out = pl.pallas_call(
        _mlp_body,
        grid=(nb,),
        in_specs=[
            blk(_BLK, 128), blk(_BLK, 128), blk(_BLK, 1), blk(_BLK, 1),
            full(b1r), full(W2), full(b2r), full(w3r), full(b3r),
        ],
        out_specs=blk(_BLK, 1),
        out_shape=jax.ShapeDtypeStruct((_B, 1), jnp.float32),
    )(u_g, i_g, uix, iix, b1r, W2, b2r, w3r, b3r)
    return out.reshape(-1)


def kernel(u_idx, i_idx, u_table, i_table, W1, b1, W2, b2, W3, b3):
    yu, yi = _tc_transform(u_table, i_table, W1[:_D, :], W1[_D:, :])
    u_g, i_g = _sc_gather_lines(u_idx, i_idx, yu, yi)
    return _tc_mlp(u_g, i_g, u_idx, i_idx, b1, W2, b2, W3, b3)


# speedup vs baseline: 4.3359x; 3.3206x over previous
import functools

import jax
import jax.numpy as jnp
from jax import lax
from jax.experimental import pallas as pl
from jax.experimental.pallas import tpu as pltpu
from jax.experimental.pallas import tpu_sc as plsc

_B = 16384
_D = 16
_N = 1000000
_PLANES = 8
_NLINES = 131072
_LB = 1024
_CHUNK = 128
_BLK = 4096


def _transform_body(*refs):
    xu = refs[:_PLANES]
    xi = refs[_PLANES:2 * _PLANES]
    wut_ref, wit_ref, yu_ref, yi_ref = refs[2 * _PLANES:]
    dn = (((1,), (0,)), ((), ()))

    def pack(wt_ref, planes):
        z = jnp.concatenate(
            [lax.dot_general(wt_ref[...], p[...], dn,
                             preferred_element_type=jnp.float32)
             for p in planes], axis=0)
        return lax.transpose(z, (1, 0))

    yu_ref[...] = pack(wut_ref, xu)
    yi_ref[...] = pack(wit_ref, xi)


def _tc_transform(u_table, i_table, w1ut, w1it):
    xu = u_table.T
    xi = i_table.T
    n_blocks = _NLINES // _LB
    bpp = _NLINES // _LB
    last_blk = (_N - 1) // _LB

    def spec(s):
        return pl.BlockSpec(
            (_D, _LB), lambda b, s=s: (0, jnp.minimum(s * bpp + b, last_blk)))

    yu, yi = pl.pallas_call(
        _transform_body,
        grid=(n_blocks,),
        in_specs=([spec(s) for s in range(_PLANES)]
                  + [spec(s) for s in range(_PLANES)]
                  + [pl.BlockSpec((_D, _D), lambda b: (0, 0))] * 2),
        out_specs=[pl.BlockSpec((_LB, 128), lambda b: (b, 0))] * 2,
        out_shape=[jax.ShapeDtypeStruct((_NLINES, 128), jnp.float32)] * 2,
    )(*([xu] * _PLANES + [xi] * _PLANES + [w1ut, w1it]))
    return yu, yi


def _sc_gather_lines(u_idx, i_idx, u_lines, i_lines):
    info = plsc.get_sparse_core_info()
    nw = info.num_cores * info.num_subcores
    b_per_w = _B // nw
    n_chunks = b_per_w // _CHUNK
    n_grp = _CHUNK // info.num_lanes

    u_idx_r = u_idx.reshape(nw, n_chunks, _CHUNK)
    i_idx_r = i_idx.reshape(nw, n_chunks, _CHUNK)

    mesh = plsc.VectorSubcoreMesh(core_axis_name="c", subcore_axis_name="s")
    out_sds = jax.ShapeDtypeStruct((nw, n_chunks, _CHUNK, 128), jnp.float32)

    @functools.partial(
        pl.kernel,
        mesh=mesh,
        out_type=(out_sds, out_sds),
        scratch_types=[
            pltpu.VMEM((n_chunks, _CHUNK), jnp.int32),
            pltpu.VMEM((n_chunks, _CHUNK), jnp.int32),
            pltpu.VMEM((n_chunks, _CHUNK), jnp.int32),
            pltpu.VMEM((n_chunks, _CHUNK), jnp.int32),
            pltpu.VMEM((2, _CHUNK, 128), jnp.float32),
            pltpu.VMEM((2, _CHUNK, 128), jnp.float32),
            pltpu.SemaphoreType.DMA,
            pltpu.SemaphoreType.DMA,
            pltpu.SemaphoreType.DMA,
            pltpu.SemaphoreType.DMA,
        ],
    )
    def gather_kernel(u_tab, i_tab, u_idx_hbm, i_idx_hbm, u_out, i_out,
                      uidx_v, iidx_v, uline_v, iline_v, ubuf, ibuf,
                      sem_u0, sem_u1, sem_i0, sem_i1):
        wid = lax.axis_index("s") * info.num_cores + lax.axis_index("c")
        pltpu.sync_copy(u_idx_hbm.at[wid], uidx_v)
        pltpu.sync_copy(i_idx_hbm.at[wid], iidx_v)
        L = info.num_lanes
        for j in range(n_chunks):
            for g in range(n_grp):
                sl = pl.ds(g * L, L)
                uline_v[j, sl] = lax.bitwise_and(uidx_v[j, sl], _NLINES - 1)
                iline_v[j, sl] = lax.bitwise_and(iidx_v[j, sl], _NLINES - 1)
        sems_u = (sem_u0, sem_u1)
        sems_i = (sem_i0, sem_i1)

        def fire(j):
            p = j % 2
            cu = pltpu.async_copy(u_tab.at[uline_v.at[j]], ubuf.at[p], sems_u[p])
            ci = pltpu.async_copy(i_tab.at[iline_v.at[j]], ibuf.at[p], sems_i[p])
            return cu, ci

        pending = fire(0)
        for j in range(n_chunks):
            cu, ci = pending
            cu.wait()
            ci.wait()
            if j + 1 < n_chunks:
                pending = fire(j + 1)
            p = j % 2
            pltpu.sync_copy(ubuf.at[p], u_out.at[wid, j])
            pltpu.sync_copy(ibuf.at[p], i_out.at[wid, j])

    u_g, i_g = gather_kernel(u_lines, i_lines, u_idx_r, i_idx_r)
    return u_g.reshape(_B, 128), i_g.reshape(_B, 128)


def _mlp_body(u_ref, i_ref, ohu_ref, ohi_ref, e_ref, s_ref, b1_ref,
              w2_ref, b2_ref, w3_ref, b3_ref, o_ref):
    dn = (((1,), (0,)), ((), ()))

    def pick(g_ref, oh_ref):
        mask = lax.dot_general(oh_ref[...], e_ref[...], dn,
                               preferred_element_type=jnp.float32)
        return lax.dot_general(g_ref[...] * mask, s_ref[...], dn,
                               preferred_element_type=jnp.float32)

    h = pick(u_ref, ohu_ref) + pick(i_ref, ohi_ref) + b1_ref[...]
    h = jnp.maximum(h, 0.0)
    h = jnp.dot(h, w2_ref[...], preferred_element_type=jnp.float32) + b2_ref[...]
    h = jnp.maximum(h, 0.0)
    z = jnp.sum(h * w3_ref[...], axis=1, keepdims=True) + b3_ref[...]
    o_ref[...] = 1.0 / (1.0 + jnp.exp(-z))


def _tc_mlp(u_g, i_g, u_idx, i_idx, b1, W2, b2, W3, b3):
    oh_u = jax.nn.one_hot(lax.shift_right_logical(u_idx, 17), _PLANES,
                          dtype=jnp.float32)
    oh_i = jax.nn.one_hot(lax.shift_right_logical(i_idx, 17), _PLANES,
                          dtype=jnp.float32)
    emat = jnp.repeat(jnp.eye(_PLANES, dtype=jnp.float32), _D, axis=1)
    smat = jnp.tile(jnp.eye(_D, dtype=jnp.float32), (_PLANES, 1))
    b1r = b1.reshape(1, -1)
    b2r = b2.reshape(1, -1)
    w3r = W3.reshape(1, -1)
    b3r = b3.reshape(1, 1)
    nb = _B // _BLK
    blk = lambda r, c: pl.BlockSpec((r, c), lambda b: (b, 0))
    full = lambda a: pl.BlockSpec(a.shape, lambda b: (0,) * a.ndim)
    out = pl.pallas_call(
        _mlp_body,
        grid=(nb,),
        in_specs=[
            blk(_BLK, 128), blk(_BLK, 128),
            blk(_BLK, _PLANES), blk(_BLK, _PLANES),
            full(emat), full(smat),
            full(b1r), full(W2), full(b2r), full(w3r), full(b3r),
        ],
        out_specs=blk(_BLK, 1),
        out_shape=jax.ShapeDtypeStruct((_B, 1), jnp.float32),
    )(u_g, i_g, oh_u, oh_i, emat, smat, b1r, W2, b2r, w3r, b3r)
    return out.reshape(-1)


def kernel(u_idx, i_idx, u_table, i_table, W1, b1, W2, b2, W3, b3):
    yu, yi = _tc_transform(u_table, i_table, W1[:_D, :].T, W1[_D:, :].T)
    u_g, i_g = _sc_gather_lines(u_idx, i_idx, yu, yi)
    return _tc_mlp(u_g, i_g, u_idx, i_idx, b1, W2, b2, W3, b3)


# speedup vs baseline: 5.4091x; 1.2475x over previous
import functools

import jax
import jax.numpy as jnp
from jax import lax
from jax.experimental import pallas as pl
from jax.experimental.pallas import tpu as pltpu
from jax.experimental.pallas import tpu_sc as plsc

_B = 16384
_D = 16
_N = 1000000
_PLANES = 8
_NLINES = 131072
_LB = 2048
_CHUNK = 128
_BLK = 4096


def _transform_body(*refs):
    xu = refs[:_PLANES]
    xi = refs[_PLANES:2 * _PLANES]
    wut_ref, wit_ref, yu_ref, yi_ref = refs[2 * _PLANES:]
    dn = (((1,), (0,)), ((), ()))

    def pack(wt_ref, planes):
        z = jnp.concatenate(
            [lax.dot_general(wt_ref[...], p[...], dn,
                             preferred_element_type=jnp.float32)
             for p in planes], axis=0)
        return lax.transpose(z, (1, 0))

    yu_ref[...] = pack(wut_ref, xu)
    yi_ref[...] = pack(wit_ref, xi)


def _tc_transform(u_table, i_table, w1ut, w1it):
    xu = u_table.T
    xi = i_table.T
    n_blocks = _NLINES // _LB
    bpp = _NLINES // _LB
    last_blk = (_N - 1) // _LB

    def spec(s):
        return pl.BlockSpec(
            (_D, _LB), lambda b, s=s: (0, jnp.minimum(s * bpp + b, last_blk)))

    yu, yi = pl.pallas_call(
        _transform_body,
        grid=(n_blocks,),
        in_specs=([spec(s) for s in range(_PLANES)]
                  + [spec(s) for s in range(_PLANES)]
                  + [pl.BlockSpec((_D, _D), lambda b: (0, 0))] * 2),
        out_specs=[pl.BlockSpec((_LB, 128), lambda b: (b, 0))] * 2,
        out_shape=[jax.ShapeDtypeStruct((_NLINES, 128), jnp.float32)] * 2,
    )(*([xu] * _PLANES + [xi] * _PLANES + [w1ut, w1it]))
    return yu, yi


def _sc_gather_lines(u_idx, i_idx, u_lines, i_lines):
    info = plsc.get_sparse_core_info()
    nw = info.num_cores * info.num_subcores
    b_per_w = _B // nw
    n_chunks = b_per_w // _CHUNK
    n_grp = _CHUNK // info.num_lanes

    u_idx_r = u_idx.reshape(nw, n_chunks, _CHUNK)
    i_idx_r = i_idx.reshape(nw, n_chunks, _CHUNK)

    mesh = plsc.VectorSubcoreMesh(core_axis_name="c", subcore_axis_name="s")
    out_sds = jax.ShapeDtypeStruct((nw, n_chunks, _CHUNK, 128), jnp.float32)

    @functools.partial(
        pl.kernel,
        mesh=mesh,
        out_type=(out_sds, out_sds),
        scratch_types=[
            pltpu.VMEM((n_chunks, _CHUNK), jnp.int32),
            pltpu.VMEM((n_chunks, _CHUNK), jnp.int32),
            pltpu.VMEM((n_chunks, _CHUNK), jnp.int32),
            pltpu.VMEM((n_chunks, _CHUNK), jnp.int32),
            pltpu.VMEM((2, _CHUNK, 128), jnp.float32),
            pltpu.VMEM((2, _CHUNK, 128), jnp.float32),
            pltpu.SemaphoreType.DMA,
            pltpu.SemaphoreType.DMA,
            pltpu.SemaphoreType.DMA,
            pltpu.SemaphoreType.DMA,
        ],
    )
    def gather_kernel(u_tab, i_tab, u_idx_hbm, i_idx_hbm, u_out, i_out,
                      uidx_v, iidx_v, uline_v, iline_v, ubuf, ibuf,
                      sem_u0, sem_u1, sem_i0, sem_i1):
        wid = lax.axis_index("s") * info.num_cores + lax.axis_index("c")
        pltpu.sync_copy(u_idx_hbm.at[wid], uidx_v)
        pltpu.sync_copy(i_idx_hbm.at[wid], iidx_v)
        L = info.num_lanes
        for j in range(n_chunks):
            for g in range(n_grp):
                sl = pl.ds(g * L, L)
                uline_v[j, sl] = lax.bitwise_and(uidx_v[j, sl], _NLINES - 1)
                iline_v[j, sl] = lax.bitwise_and(iidx_v[j, sl], _NLINES - 1)
        sems_u = (sem_u0, sem_u1)
        sems_i = (sem_i0, sem_i1)

        def fire(j):
            p = j % 2
            cu = pltpu.async_copy(u_tab.at[uline_v.at[j]], ubuf.at[p], sems_u[p])
            ci = pltpu.async_copy(i_tab.at[iline_v.at[j]], ibuf.at[p], sems_i[p])
            return cu, ci

        pending = fire(0)
        for j in range(n_chunks):
            cu, ci = pending
            cu.wait()
            ci.wait()
            if j + 1 < n_chunks:
                pending = fire(j + 1)
            p = j % 2
            pltpu.sync_copy(ubuf.at[p], u_out.at[wid, j])
            pltpu.sync_copy(ibuf.at[p], i_out.at[wid, j])

    u_g, i_g = gather_kernel(u_lines, i_lines, u_idx_r, i_idx_r)
    return u_g.reshape(_B, 128), i_g.reshape(_B, 128)


def _mlp_body(u_ref, i_ref, ohu_ref, ohi_ref, e_ref, s_ref, b1_ref,
              w2_ref, b2_ref, w3_ref, b3_ref, o_ref):
    dn = (((1,), (0,)), ((), ()))

    def pick(g_ref, oh_ref):
        mask = lax.dot_general(oh_ref[...], e_ref[...], dn,
                               preferred_element_type=jnp.float32)
        return lax.dot_general(g_ref[...] * mask, s_ref[...], dn,
                               preferred_element_type=jnp.float32)

    h = pick(u_ref, ohu_ref) + pick(i_ref, ohi_ref) + b1_ref[...]
    h = jnp.maximum(h, 0.0)
    h = jnp.dot(h, w2_ref[...], preferred_element_type=jnp.float32) + b2_ref[...]
    h = jnp.maximum(h, 0.0)
    z = jnp.sum(h * w3_ref[...], axis=1, keepdims=True) + b3_ref[...]
    o_ref[...] = 1.0 / (1.0 + jnp.exp(-z))


def _tc_mlp(u_g, i_g, u_idx, i_idx, b1, W2, b2, W3, b3):
    oh_u = jax.nn.one_hot(lax.shift_right_logical(u_idx, 17), _PLANES,
                          dtype=jnp.float32)
    oh_i = jax.nn.one_hot(lax.shift_right_logical(i_idx, 17), _PLANES,
                          dtype=jnp.float32)
    emat = jnp.repeat(jnp.eye(_PLANES, dtype=jnp.float32), _D, axis=1)
    smat = jnp.tile(jnp.eye(_D, dtype=jnp.float32), (_PLANES, 1))
    b1r = b1.reshape(1, -1)
    b2r = b2.reshape(1, -1)
    w3r = W3.reshape(1, -1)
    b3r = b3.reshape(1, 1)
    nb = _B // _BLK
    blk = lambda r, c: pl.BlockSpec((r, c), lambda b: (b, 0))
    full = lambda a: pl.BlockSpec(a.shape, lambda b: (0,) * a.ndim)
    out = pl.pallas_call(
        _mlp_body,
        grid=(nb,),
        in_specs=[
            blk(_BLK, 128), blk(_BLK, 128),
            blk(_BLK, _PLANES), blk(_BLK, _PLANES),
            full(emat), full(smat),
            full(b1r), full(W2), full(b2r), full(w3r), full(b3r),
        ],
        out_specs=blk(_BLK, 1),
        out_shape=jax.ShapeDtypeStruct((_B, 1), jnp.float32),
    )(u_g, i_g, oh_u, oh_i, emat, smat, b1r, W2, b2r, w3r, b3r)
    return out.reshape(-1)


def kernel(u_idx, i_idx, u_table, i_table, W1, b1, W2, b2, W3, b3):
    yu, yi = _tc_transform(u_table, i_table, W1[:_D, :].T, W1[_D:, :].T)
    u_g, i_g = _sc_gather_lines(u_idx, i_idx, yu, yi)
    return _tc_mlp(u_g, i_g, u_idx, i_idx, b1, W2, b2, W3, b3)


# speedup vs baseline: 6.1705x; 1.1408x over previous
import functools

import jax
import jax.numpy as jnp
from jax import lax
from jax.experimental import pallas as pl
from jax.experimental.pallas import tpu as pltpu
from jax.experimental.pallas import tpu_sc as plsc

_B = 16384
_D = 16
_N = 1000000
_PLANES = 8
_NLINES = 131072
_LB = 4096
_CHUNK = 128
_BLK = 4096


def _transform_body(*refs):
    xu = refs[:_PLANES]
    xi = refs[_PLANES:2 * _PLANES]
    wut_ref, wit_ref, yu_ref, yi_ref = refs[2 * _PLANES:]
    dn = (((1,), (0,)), ((), ()))

    def pack(wt_ref, planes):
        z = jnp.concatenate(
            [lax.dot_general(wt_ref[...], p[...], dn,
                             preferred_element_type=jnp.float32)
             for p in planes], axis=0)
        return lax.transpose(z, (1, 0))

    yu_ref[...] = pack(wut_ref, xu)
    yi_ref[...] = pack(wit_ref, xi)


def _tc_transform(u_table, i_table, w1ut, w1it):
    xu = u_table.T
    xi = i_table.T
    n_blocks = _NLINES // _LB
    bpp = _NLINES // _LB
    last_blk = (_N - 1) // _LB

    def spec(s):
        return pl.BlockSpec(
            (_D, _LB), lambda b, s=s: (0, jnp.minimum(s * bpp + b, last_blk)))

    yu, yi = pl.pallas_call(
        _transform_body,
        grid=(n_blocks,),
        in_specs=([spec(s) for s in range(_PLANES)]
                  + [spec(s) for s in range(_PLANES)]
                  + [pl.BlockSpec((_D, _D), lambda b: (0, 0))] * 2),
        out_specs=[pl.BlockSpec((_LB, 128), lambda b: (b, 0))] * 2,
        out_shape=[jax.ShapeDtypeStruct((_NLINES, 128), jnp.float32)] * 2,
    )(*([xu] * _PLANES + [xi] * _PLANES + [w1ut, w1it]))
    return yu, yi


def _sc_gather_lines(u_idx, i_idx, u_lines, i_lines):
    info = plsc.get_sparse_core_info()
    nw = info.num_cores * info.num_subcores
    b_per_w = _B // nw
    n_chunks = b_per_w // _CHUNK
    n_grp = _CHUNK // info.num_lanes

    u_idx_r = u_idx.reshape(nw, n_chunks, _CHUNK)
    i_idx_r = i_idx.reshape(nw, n_chunks, _CHUNK)

    mesh = plsc.VectorSubcoreMesh(core_axis_name="c", subcore_axis_name="s")
    out_sds = jax.ShapeDtypeStruct((nw, n_chunks, _CHUNK, 128), jnp.float32)

    @functools.partial(
        pl.kernel,
        mesh=mesh,
        out_type=(out_sds, out_sds),
        scratch_types=[
            pltpu.VMEM((n_chunks, _CHUNK), jnp.int32),
            pltpu.VMEM((n_chunks, _CHUNK), jnp.int32),
            pltpu.VMEM((n_chunks, _CHUNK), jnp.int32),
            pltpu.VMEM((n_chunks, _CHUNK), jnp.int32),
            pltpu.VMEM((2, _CHUNK, 128), jnp.float32),
            pltpu.VMEM((2, _CHUNK, 128), jnp.float32),
            pltpu.SemaphoreType.DMA,
            pltpu.SemaphoreType.DMA,
            pltpu.SemaphoreType.DMA,
            pltpu.SemaphoreType.DMA,
        ],
    )
    def gather_kernel(u_tab, i_tab, u_idx_hbm, i_idx_hbm, u_out, i_out,
                      uidx_v, iidx_v, uline_v, iline_v, ubuf, ibuf,
                      sem_u0, sem_u1, sem_i0, sem_i1):
        wid = lax.axis_index("s") * info.num_cores + lax.axis_index("c")
        pltpu.sync_copy(u_idx_hbm.at[wid], uidx_v)
        pltpu.sync_copy(i_idx_hbm.at[wid], iidx_v)
        L = info.num_lanes
        for j in range(n_chunks):
            for g in range(n_grp):
                sl = pl.ds(g * L, L)
                uline_v[j, sl] = lax.bitwise_and(uidx_v[j, sl], _NLINES - 1)
                iline_v[j, sl] = lax.bitwise_and(iidx_v[j, sl], _NLINES - 1)
        sems_u = (sem_u0, sem_u1)
        sems_i = (sem_i0, sem_i1)

        def fire(j):
            p = j % 2
            cu = pltpu.async_copy(u_tab.at[uline_v.at[j]], ubuf.at[p], sems_u[p])
            ci = pltpu.async_copy(i_tab.at[iline_v.at[j]], ibuf.at[p], sems_i[p])
            return cu, ci

        pending = fire(0)
        for j in range(n_chunks):
            cu, ci = pending
            cu.wait()
            ci.wait()
            if j + 1 < n_chunks:
                pending = fire(j + 1)
            p = j % 2
            pltpu.sync_copy(ubuf.at[p], u_out.at[wid, j])
            pltpu.sync_copy(ibuf.at[p], i_out.at[wid, j])

    u_g, i_g = gather_kernel(u_lines, i_lines, u_idx_r, i_idx_r)
    return u_g.reshape(_B, 128), i_g.reshape(_B, 128)


def _mlp_body(u_ref, i_ref, ohu_ref, ohi_ref, e_ref, s_ref, b1_ref,
              w2_ref, b2_ref, w3_ref, b3_ref, o_ref):
    dn = (((1,), (0,)), ((), ()))

    def pick(g_ref, oh_ref):
        mask = lax.dot_general(oh_ref[...], e_ref[...], dn,
                               preferred_element_type=jnp.float32)
        return lax.dot_general(g_ref[...] * mask, s_ref[...], dn,
                               preferred_element_type=jnp.float32)

    h = pick(u_ref, ohu_ref) + pick(i_ref, ohi_ref) + b1_ref[...]
    h = jnp.maximum(h, 0.0)
    h = jnp.dot(h, w2_ref[...], preferred_element_type=jnp.float32) + b2_ref[...]
    h = jnp.maximum(h, 0.0)
    z = jnp.sum(h * w3_ref[...], axis=1, keepdims=True) + b3_ref[...]
    o_ref[...] = 1.0 / (1.0 + jnp.exp(-z))


def _tc_mlp(u_g, i_g, u_idx, i_idx, b1, W2, b2, W3, b3):
    oh_u = jax.nn.one_hot(lax.shift_right_logical(u_idx, 17), _PLANES,
                          dtype=jnp.float32)
    oh_i = jax.nn.one_hot(lax.shift_right_logical(i_idx, 17), _PLANES,
                          dtype=jnp.float32)
    emat = jnp.repeat(jnp.eye(_PLANES, dtype=jnp.float32), _D, axis=1)
    smat = jnp.tile(jnp.eye(_D, dtype=jnp.float32), (_PLANES, 1))
    b1r = b1.reshape(1, -1)
    b2r = b2.reshape(1, -1)
    w3r = W3.reshape(1, -1)
    b3r = b3.reshape(1, 1)
    nb = _B // _BLK
    blk = lambda r, c: pl.BlockSpec((r, c), lambda b: (b, 0))
    full = lambda a: pl.BlockSpec(a.shape, lambda b: (0,) * a.ndim)
    out = pl.pallas_call(
        _mlp_body,
        grid=(nb,),
        in_specs=[
            blk(_BLK, 128), blk(_BLK, 128),
            blk(_BLK, _PLANES), blk(_BLK, _PLANES),
            full(emat), full(smat),
            full(b1r), full(W2), full(b2r), full(w3r), full(b3r),
        ],
        out_specs=blk(_BLK, 1),
        out_shape=jax.ShapeDtypeStruct((_B, 1), jnp.float32),
    )(u_g, i_g, oh_u, oh_i, emat, smat, b1r, W2, b2r, w3r, b3r)
    return out.reshape(-1)


def kernel(u_idx, i_idx, u_table, i_table, W1, b1, W2, b2, W3, b3):
    yu, yi = _tc_transform(u_table, i_table, W1[:_D, :].T, W1[_D:, :].T)
    u_g, i_g = _sc_gather_lines(u_idx, i_idx, yu, yi)
    return _tc_mlp(u_g, i_g, u_idx, i_idx, b1, W2, b2, W3, b3)


# speedup vs baseline: 6.3988x; 1.0370x over previous
import functools

import jax
import jax.numpy as jnp
from jax import lax
from jax.experimental import pallas as pl
from jax.experimental.pallas import tpu as pltpu
from jax.experimental.pallas import tpu_sc as plsc

_B = 16384
_D = 16
_N = 1000000
_PLANES = 8
_NLINES = 131072
_LB = 8192
_CHUNK = 128
_BLK = 4096


def _transform_body(*refs):
    xu = refs[:_PLANES]
    xi = refs[_PLANES:2 * _PLANES]
    wut_ref, wit_ref, yu_ref, yi_ref = refs[2 * _PLANES:]
    dn = (((1,), (0,)), ((), ()))

    def pack(wt_ref, planes):
        z = jnp.concatenate(
            [lax.dot_general(wt_ref[...], p[...], dn,
                             preferred_element_type=jnp.float32)
             for p in planes], axis=0)
        return lax.transpose(z, (1, 0))

    yu_ref[...] = pack(wut_ref, xu)
    yi_ref[...] = pack(wit_ref, xi)


def _tc_transform(u_table, i_table, w1ut, w1it):
    xu = u_table.T
    xi = i_table.T
    n_blocks = _NLINES // _LB
    bpp = _NLINES // _LB
    last_blk = (_N - 1) // _LB

    def spec(s):
        return pl.BlockSpec(
            (_D, _LB), lambda b, s=s: (0, jnp.minimum(s * bpp + b, last_blk)))

    yu, yi = pl.pallas_call(
        _transform_body,
        grid=(n_blocks,),
        in_specs=([spec(s) for s in range(_PLANES)]
                  + [spec(s) for s in range(_PLANES)]
                  + [pl.BlockSpec((_D, _D), lambda b: (0, 0))] * 2),
        out_specs=[pl.BlockSpec((_LB, 128), lambda b: (b, 0))] * 2,
        out_shape=[jax.ShapeDtypeStruct((_NLINES, 128), jnp.float32)] * 2,
    )(*([xu] * _PLANES + [xi] * _PLANES + [w1ut, w1it]))
    return yu, yi


def _sc_gather_lines(u_idx, i_idx, u_lines, i_lines):
    info = plsc.get_sparse_core_info()
    nw = info.num_cores * info.num_subcores
    b_per_w = _B // nw
    n_chunks = b_per_w // _CHUNK
    n_grp = _CHUNK // info.num_lanes

    u_idx_r = u_idx.reshape(nw, n_chunks, _CHUNK)
    i_idx_r = i_idx.reshape(nw, n_chunks, _CHUNK)

    mesh = plsc.VectorSubcoreMesh(core_axis_name="c", subcore_axis_name="s")
    out_sds = jax.ShapeDtypeStruct((nw, n_chunks, _CHUNK, 128), jnp.float32)

    @functools.partial(
        pl.kernel,
        mesh=mesh,
        out_type=(out_sds, out_sds),
        scratch_types=[
            pltpu.VMEM((n_chunks, _CHUNK), jnp.int32),
            pltpu.VMEM((n_chunks, _CHUNK), jnp.int32),
            pltpu.VMEM((n_chunks, _CHUNK), jnp.int32),
            pltpu.VMEM((n_chunks, _CHUNK), jnp.int32),
            pltpu.VMEM((2, _CHUNK, 128), jnp.float32),
            pltpu.VMEM((2, _CHUNK, 128), jnp.float32),
            pltpu.SemaphoreType.DMA,
            pltpu.SemaphoreType.DMA,
            pltpu.SemaphoreType.DMA,
            pltpu.SemaphoreType.DMA,
        ],
    )
    def gather_kernel(u_tab, i_tab, u_idx_hbm, i_idx_hbm, u_out, i_out,
                      uidx_v, iidx_v, uline_v, iline_v, ubuf, ibuf,
                      sem_u0, sem_u1, sem_i0, sem_i1):
        wid = lax.axis_index("s") * info.num_cores + lax.axis_index("c")
        pltpu.sync_copy(u_idx_hbm.at[wid], uidx_v)
        pltpu.sync_copy(i_idx_hbm.at[wid], iidx_v)
        L = info.num_lanes
        for j in range(n_chunks):
            for g in range(n_grp):
                sl = pl.ds(g * L, L)
                uline_v[j, sl] = lax.bitwise_and(uidx_v[j, sl], _NLINES - 1)
                iline_v[j, sl] = lax.bitwise_and(iidx_v[j, sl], _NLINES - 1)
        sems_u = (sem_u0, sem_u1)
        sems_i = (sem_i0, sem_i1)

        def fire(j):
            p = j % 2
            cu = pltpu.async_copy(u_tab.at[uline_v.at[j]], ubuf.at[p], sems_u[p])
            ci = pltpu.async_copy(i_tab.at[iline_v.at[j]], ibuf.at[p], sems_i[p])
            return cu, ci

        pending = fire(0)
        for j in range(n_chunks):
            cu, ci = pending
            cu.wait()
            ci.wait()
            if j + 1 < n_chunks:
                pending = fire(j + 1)
            p = j % 2
            pltpu.sync_copy(ubuf.at[p], u_out.at[wid, j])
            pltpu.sync_copy(ibuf.at[p], i_out.at[wid, j])

    u_g, i_g = gather_kernel(u_lines, i_lines, u_idx_r, i_idx_r)
    return u_g.reshape(_B, 128), i_g.reshape(_B, 128)


def _mlp_body(u_ref, i_ref, ohu_ref, ohi_ref, e_ref, s_ref, b1_ref,
              w2_ref, b2_ref, w3_ref, b3_ref, o_ref):
    dn = (((1,), (0,)), ((), ()))

    def pick(g_ref, oh_ref):
        mask = lax.dot_general(oh_ref[...], e_ref[...], dn,
                               preferred_element_type=jnp.float32)
        return lax.dot_general(g_ref[...] * mask, s_ref[...], dn,
                               preferred_element_type=jnp.float32)

    h = pick(u_ref, ohu_ref) + pick(i_ref, ohi_ref) + b1_ref[...]
    h = jnp.maximum(h, 0.0)
    h = jnp.dot(h, w2_ref[...], preferred_element_type=jnp.float32) + b2_ref[...]
    h = jnp.maximum(h, 0.0)
    z = jnp.sum(h * w3_ref[...], axis=1, keepdims=True) + b3_ref[...]
    o_ref[...] = 1.0 / (1.0 + jnp.exp(-z))


def _tc_mlp(u_g, i_g, u_idx, i_idx, b1, W2, b2, W3, b3):
    oh_u = jax.nn.one_hot(lax.shift_right_logical(u_idx, 17), _PLANES,
                          dtype=jnp.float32)
    oh_i = jax.nn.one_hot(lax.shift_right_logical(i_idx, 17), _PLANES,
                          dtype=jnp.float32)
    emat = jnp.repeat(jnp.eye(_PLANES, dtype=jnp.float32), _D, axis=1)
    smat = jnp.tile(jnp.eye(_D, dtype=jnp.float32), (_PLANES, 1))
    b1r = b1.reshape(1, -1)
    b2r = b2.reshape(1, -1)
    w3r = W3.reshape(1, -1)
    b3r = b3.reshape(1, 1)
    nb = _B // _BLK
    blk = lambda r, c: pl.BlockSpec((r, c), lambda b: (b, 0))
    full = lambda a: pl.BlockSpec(a.shape, lambda b: (0,) * a.ndim)
    out = pl.pallas_call(
        _mlp_body,
        grid=(nb,),
        in_specs=[
            blk(_BLK, 128), blk(_BLK, 128),
            blk(_BLK, _PLANES), blk(_BLK, _PLANES),
            full(emat), full(smat),
            full(b1r), full(W2), full(b2r), full(w3r), full(b3r),
        ],
        out_specs=blk(_BLK, 1),
        out_shape=jax.ShapeDtypeStruct((_B, 1), jnp.float32),
    )(u_g, i_g, oh_u, oh_i, emat, smat, b1r, W2, b2r, w3r, b3r)
    return out.reshape(-1)


def kernel(u_idx, i_idx, u_table, i_table, W1, b1, W2, b2, W3, b3):
    yu, yi = _tc_transform(u_table, i_table, W1[:_D, :].T, W1[_D:, :].T)
    u_g, i_g = _sc_gather_lines(u_idx, i_idx, yu, yi)
    return _tc_mlp(u_g, i_g, u_idx, i_idx, b1, W2, b2, W3, b3)
